# baseline (speedup 1.0000x reference)
import functools

import jax
import jax.numpy as jnp
from jax import lax
from jax.experimental import pallas as pl
from jax.experimental.pallas import tpu as pltpu
from jax.experimental.pallas import tpu_sc as plsc

N = 4096
M = 4096
K = 4096

BN = 256
SC_ROWS = 1024
SC_BLOCKS = SC_ROWS // BN

NC = 2
NS = 16
NW = NC * NS
L = 16
RPW = SC_ROWS // NW
G = 4
NG = RPW // G
KC = K // L


def _tc_kernel(x_ref, y_ref, Q_ref, AT_ref, c_ref, out_ref, acc_ref):
    i = pl.program_id(0)

    @pl.when(i == 0)
    def _init():
        c_full = c_ref[...]
        acc_ref[0, 0] = 0.0
        acc_ref[0, 1] = jnp.sum(c_full * c_full)

    c_blk = c_ref[pl.ds((SC_BLOCKS + i) * BN, BN), :]
    r = (
        jnp.dot(Q_ref[...], x_ref[...], preferred_element_type=jnp.float32)
        + jnp.dot(AT_ref[...], y_ref[...], preferred_element_type=jnp.float32)
        + c_blk
    )
    acc_ref[0, 0] += jnp.sum(r * r)

    @pl.when(i == pl.num_programs(0) - 1)
    def _fin():
        out_ref[...] = jnp.concatenate(
            [
                jnp.full((1, 1), acc_ref[0, 0], dtype=jnp.float32),
                jnp.full((1, 1), acc_ref[0, 1], dtype=jnp.float32),
            ],
            axis=1,
        )


def _tc_partial(Q, AT, c2, x, y):
    n_tc = N - SC_ROWS
    return pl.pallas_call(
        _tc_kernel,
        grid=(n_tc // BN,),
        in_specs=[
            pl.BlockSpec((K, 1), lambda i: (0, 0)),
            pl.BlockSpec((K, 1), lambda i: (0, 0)),
            pl.BlockSpec((BN, K), lambda i: (SC_BLOCKS + i, 0)),
            pl.BlockSpec((BN, K), lambda i: (SC_BLOCKS + i, 0)),
            pl.BlockSpec((N, 1), lambda i: (0, 0)),
        ],
        out_specs=pl.BlockSpec((1, 2), lambda i: (0, 0)),
        out_shape=jax.ShapeDtypeStruct((1, 2), jnp.float32),
        scratch_shapes=[pltpu.SMEM((1, 2), jnp.float32)],
    )(x, y, Q, AT, c2)


def _hsum(v, tmp_ref):
    idx = lax.iota(jnp.int32, L)
    for s in (8, 4, 2, 1):
        tmp_ref[...] = v
        perm = (idx + s) & (L - 1)
        v = v + plsc.load_gather(tmp_ref, [perm])
    return v[0]


def _sc_body(Q_hbm, AT_hbm, c_hbm, x_hbm, y_hbm, out_hbm,
             x_v, y_v, c_v, q_v, a_v, o_v, tmp_v,
             sx, sy, sc_, sq0, sq1, sa0, sa1):
    wid = lax.axis_index("s") * NC + lax.axis_index("c")
    base = wid * RPW
    cp_x = pltpu.async_copy(x_hbm, x_v, sx)
    cp_y = pltpu.async_copy(y_hbm, y_v, sy)
    cp_c = pltpu.async_copy(c_hbm.at[pl.ds(base, RPW)], c_v, sc_)

    q_sems = (sq0, sq1)
    a_sems = (sa0, sa1)

    def start(g):
        buf = g % 2
        row0 = base + g * G
        hq = pltpu.async_copy(Q_hbm.at[pl.ds(row0, G)], q_v.at[buf], q_sems[buf])
        ha = pltpu.async_copy(AT_hbm.at[pl.ds(row0, G)], a_v.at[buf], a_sems[buf])
        return hq, ha

    handles = [None] * NG
    handles[0] = start(0)

    cp_x.wait()
    cp_y.wait()
    cp_c.wait()
    c_lo = c_v[pl.ds(0, L)]
    c_hi = c_v[pl.ds(L, L)]

    acc = jnp.float32(0.0)
    for g in range(NG):
        buf = g % 2
        if g + 1 < NG:
            handles[g + 1] = start(g + 1)
        hq, ha = handles[g]
        hq.wait()
        ha.wait()
        qb = q_v.at[buf]
        ab = a_v.at[buf]

        def body(i, accs):
            o = i * (2 * L)
            new = list(accs)
            for u in range(2):
                oo = o + u * L
                xk = x_v[pl.ds(oo, L)]
                yk = y_v[pl.ds(oo, L)]
                for gg in range(G):
                    new[gg] = new[gg] + qb[gg, pl.ds(oo, L)] * xk
                    new[G + gg] = new[G + gg] + ab[gg, pl.ds(oo, L)] * yk
            return tuple(new)

        zeros = tuple(jnp.zeros((L,), jnp.float32) for _ in range(2 * G))
        accs = lax.fori_loop(0, KC // 2, body, zeros)
        for gg in range(G):
            idx = g * G + gg
            c_val = c_lo[idx] if idx < L else c_hi[idx - L]
            v = _hsum(accs[gg] + accs[G + gg], tmp_v) + c_val
            acc = acc + v * v

    o_v[...] = jnp.full((L,), acc * 0.0625, dtype=jnp.float32)
    pltpu.sync_copy(o_v, out_hbm.at[wid])


def _sc_partial(Q, AT, c1, xf, yf):
    mesh = plsc.VectorSubcoreMesh(core_axis_name="c", subcore_axis_name="s")
    run = pl.kernel(
        _sc_body,
        out_type=jax.ShapeDtypeStruct((NW, L), jnp.float32),
        mesh=mesh,
        scratch_types=[
            pltpu.VMEM((K,), jnp.float32),
            pltpu.VMEM((K,), jnp.float32),
            pltpu.VMEM((RPW,), jnp.float32),
            pltpu.VMEM((2, G, K), jnp.float32),
            pltpu.VMEM((2, G, K), jnp.float32),
            pltpu.VMEM((L,), jnp.float32),
            pltpu.VMEM((L,), jnp.float32),
            pltpu.SemaphoreType.DMA,
            pltpu.SemaphoreType.DMA,
            pltpu.SemaphoreType.DMA,
            pltpu.SemaphoreType.DMA,
            pltpu.SemaphoreType.DMA,
            pltpu.SemaphoreType.DMA,
            pltpu.SemaphoreType.DMA,
        ],
        compiler_params=pltpu.CompilerParams(needs_layout_passes=False),
    )
    return run(Q, AT, c1, xf, yf)


BK = 2048
BN2 = 256


def _tc2_kernel(x_ref, y_ref, Q_ref, AT_ref, c_ref, out_ref, acc_ref, vec_ref):
    i = pl.program_id(0)
    k = pl.program_id(1)

    @pl.when(jnp.logical_and(i == 0, k == 0))
    def _init():
        c_full = c_ref[...]
        acc_ref[0, 0] = 0.0
        acc_ref[0, 1] = jnp.sum(c_full * c_full)

    part = (
        jnp.dot(Q_ref[...], x_ref[...], preferred_element_type=jnp.float32)
        + jnp.dot(AT_ref[...], y_ref[...], preferred_element_type=jnp.float32)
    )

    @pl.when(k == 0)
    def _first():
        vec_ref[...] = part

    @pl.when(k > 0)
    def _rest():
        vec_ref[...] += part

    @pl.when(k == pl.num_programs(1) - 1)
    def _row_done():
        r = vec_ref[...] + c_ref[pl.ds(i * BN2, BN2), :]
        acc_ref[0, 0] += jnp.sum(r * r)

    @pl.when(
        jnp.logical_and(i == pl.num_programs(0) - 1, k == pl.num_programs(1) - 1)
    )
    def _fin():
        out_ref[...] = jnp.concatenate(
            [
                jnp.full((1, 1), acc_ref[0, 0], dtype=jnp.float32),
                jnp.full((1, 1), acc_ref[0, 1], dtype=jnp.float32),
            ],
            axis=1,
        )


def _tc2_full(Q, AT, c2, x, y):
    return pl.pallas_call(
        _tc2_kernel,
        grid=(N // BN2, K // BK),
        in_specs=[
            pl.BlockSpec((BK, 1), lambda i, k: (k, 0)),
            pl.BlockSpec((BK, 1), lambda i, k: (k, 0)),
            pl.BlockSpec((BN2, BK), lambda i, k: (i, k)),
            pl.BlockSpec((BN2, BK), lambda i, k: (i, k)),
            pl.BlockSpec((N, 1), lambda i, k: (0, 0)),
        ],
        out_specs=pl.BlockSpec((1, 2), lambda i, k: (0, 0)),
        out_shape=jax.ShapeDtypeStruct((1, 2), jnp.float32),
        scratch_shapes=[
            pltpu.SMEM((1, 2), jnp.float32),
            pltpu.VMEM((BN2, 1), jnp.float32),
        ],
    )(x, y, Q, AT, c2)


def kernel(Q, AT, b, c, x, y):
    del b
    c2 = c.reshape(N, 1)
    tc_out = _tc2_full(Q, AT, c2, x, y)
    return jnp.sqrt(tc_out[0, 0]) / (0.0001 + jnp.sqrt(tc_out[0, 1]))


# speedup vs baseline: 1.3640x; 1.3640x over previous
import functools

import jax
import jax.numpy as jnp
from jax import lax
from jax.experimental import pallas as pl
from jax.experimental.pallas import tpu as pltpu
from jax.experimental.pallas import tpu_sc as plsc

N = 4096
M = 4096
K = 4096

BN = 256
SC_ROWS = 1024
SC_BLOCKS = SC_ROWS // BN

NC = 2
NS = 16
NW = NC * NS
L = 16
RPW = SC_ROWS // NW
G = 4
NG = RPW // G
KC = K // L


def _tc_kernel(x_ref, y_ref, Q_ref, AT_ref, c_ref, out_ref, acc_ref):
    i = pl.program_id(0)

    @pl.when(i == 0)
    def _init():
        c_full = c_ref[...]
        acc_ref[0, 0] = 0.0
        acc_ref[0, 1] = jnp.sum(c_full * c_full)

    c_blk = c_ref[pl.ds((SC_BLOCKS + i) * BN, BN), :]
    r = (
        jnp.dot(Q_ref[...], x_ref[...], preferred_element_type=jnp.float32)
        + jnp.dot(AT_ref[...], y_ref[...], preferred_element_type=jnp.float32)
        + c_blk
    )
    acc_ref[0, 0] += jnp.sum(r * r)

    @pl.when(i == pl.num_programs(0) - 1)
    def _fin():
        out_ref[...] = jnp.concatenate(
            [
                jnp.full((1, 1), acc_ref[0, 0], dtype=jnp.float32),
                jnp.full((1, 1), acc_ref[0, 1], dtype=jnp.float32),
            ],
            axis=1,
        )


def _tc_partial(Q, AT, c2, x, y):
    n_tc = N - SC_ROWS
    return pl.pallas_call(
        _tc_kernel,
        grid=(n_tc // BN,),
        in_specs=[
            pl.BlockSpec((K, 1), lambda i: (0, 0)),
            pl.BlockSpec((K, 1), lambda i: (0, 0)),
            pl.BlockSpec((BN, K), lambda i: (SC_BLOCKS + i, 0)),
            pl.BlockSpec((BN, K), lambda i: (SC_BLOCKS + i, 0)),
            pl.BlockSpec((N, 1), lambda i: (0, 0)),
        ],
        out_specs=pl.BlockSpec((1, 2), lambda i: (0, 0)),
        out_shape=jax.ShapeDtypeStruct((1, 2), jnp.float32),
        scratch_shapes=[pltpu.SMEM((1, 2), jnp.float32)],
    )(x, y, Q, AT, c2)


def _hsum(v, tmp_ref):
    idx = lax.iota(jnp.int32, L)
    for s in (8, 4, 2, 1):
        tmp_ref[...] = v
        perm = (idx + s) & (L - 1)
        v = v + plsc.load_gather(tmp_ref, [perm])
    return v[0]


def _sc_body(Q_hbm, AT_hbm, c_hbm, x_hbm, y_hbm, out_hbm,
             x_v, y_v, c_v, q_v, a_v, o_v, tmp_v,
             sx, sy, sc_, sq0, sq1, sa0, sa1):
    wid = lax.axis_index("s") * NC + lax.axis_index("c")
    base = wid * RPW
    cp_x = pltpu.async_copy(x_hbm, x_v, sx)
    cp_y = pltpu.async_copy(y_hbm, y_v, sy)
    cp_c = pltpu.async_copy(c_hbm.at[pl.ds(base, RPW)], c_v, sc_)

    q_sems = (sq0, sq1)
    a_sems = (sa0, sa1)

    def start(g):
        buf = g % 2
        row0 = base + g * G
        hq = pltpu.async_copy(Q_hbm.at[pl.ds(row0, G)], q_v.at[buf], q_sems[buf])
        ha = pltpu.async_copy(AT_hbm.at[pl.ds(row0, G)], a_v.at[buf], a_sems[buf])
        return hq, ha

    handles = [None] * NG
    handles[0] = start(0)

    cp_x.wait()
    cp_y.wait()
    cp_c.wait()
    c_lo = c_v[pl.ds(0, L)]
    c_hi = c_v[pl.ds(L, L)]

    acc = jnp.float32(0.0)
    for g in range(NG):
        buf = g % 2
        if g + 1 < NG:
            handles[g + 1] = start(g + 1)
        hq, ha = handles[g]
        hq.wait()
        ha.wait()
        qb = q_v.at[buf]
        ab = a_v.at[buf]

        def body(i, accs):
            o = i * (2 * L)
            new = list(accs)
            for u in range(2):
                oo = o + u * L
                xk = x_v[pl.ds(oo, L)]
                yk = y_v[pl.ds(oo, L)]
                for gg in range(G):
                    new[gg] = new[gg] + qb[gg, pl.ds(oo, L)] * xk
                    new[G + gg] = new[G + gg] + ab[gg, pl.ds(oo, L)] * yk
            return tuple(new)

        zeros = tuple(jnp.zeros((L,), jnp.float32) for _ in range(2 * G))
        accs = lax.fori_loop(0, KC // 2, body, zeros)
        for gg in range(G):
            idx = g * G + gg
            c_val = c_lo[idx] if idx < L else c_hi[idx - L]
            v = _hsum(accs[gg] + accs[G + gg], tmp_v) + c_val
            acc = acc + v * v

    o_v[...] = jnp.full((L,), acc * 0.0625, dtype=jnp.float32)
    pltpu.sync_copy(o_v, out_hbm.at[wid])


def _sc_partial(Q, AT, c1, xf, yf):
    mesh = plsc.VectorSubcoreMesh(core_axis_name="c", subcore_axis_name="s")
    run = pl.kernel(
        _sc_body,
        out_type=jax.ShapeDtypeStruct((NW, L), jnp.float32),
        mesh=mesh,
        scratch_types=[
            pltpu.VMEM((K,), jnp.float32),
            pltpu.VMEM((K,), jnp.float32),
            pltpu.VMEM((RPW,), jnp.float32),
            pltpu.VMEM((2, G, K), jnp.float32),
            pltpu.VMEM((2, G, K), jnp.float32),
            pltpu.VMEM((L,), jnp.float32),
            pltpu.VMEM((L,), jnp.float32),
            pltpu.SemaphoreType.DMA,
            pltpu.SemaphoreType.DMA,
            pltpu.SemaphoreType.DMA,
            pltpu.SemaphoreType.DMA,
            pltpu.SemaphoreType.DMA,
            pltpu.SemaphoreType.DMA,
            pltpu.SemaphoreType.DMA,
        ],
        compiler_params=pltpu.CompilerParams(needs_layout_passes=False),
    )
    return run(Q, AT, c1, xf, yf)


BK = 2048
BN2 = 256


def _tc2_kernel(x_ref, y_ref, Q_ref, AT_ref, c_ref, out_ref, acc_ref, vec_ref):
    i = pl.program_id(0)
    k = pl.program_id(1)

    @pl.when(jnp.logical_and(i == 0, k == 0))
    def _init():
        c_full = c_ref[...]
        acc_ref[0, 0] = 0.0
        acc_ref[0, 1] = jnp.sum(c_full * c_full)

    part = (
        jnp.dot(Q_ref[...], x_ref[...], preferred_element_type=jnp.float32)
        + jnp.dot(AT_ref[...], y_ref[...], preferred_element_type=jnp.float32)
    )

    @pl.when(k == 0)
    def _first():
        vec_ref[...] = part

    @pl.when(k > 0)
    def _rest():
        vec_ref[...] += part

    @pl.when(k == pl.num_programs(1) - 1)
    def _row_done():
        r = vec_ref[...] + c_ref[pl.ds(i * BN2, BN2), :]
        acc_ref[0, 0] += jnp.sum(r * r)

    @pl.when(
        jnp.logical_and(i == pl.num_programs(0) - 1, k == pl.num_programs(1) - 1)
    )
    def _fin():
        out_ref[...] = jnp.concatenate(
            [
                jnp.full((1, 1), acc_ref[0, 0], dtype=jnp.float32),
                jnp.full((1, 1), acc_ref[0, 1], dtype=jnp.float32),
            ],
            axis=1,
        )


def _tc2_full(Q, AT, c2, x, y):
    return pl.pallas_call(
        _tc2_kernel,
        grid=(N // BN2, K // BK),
        in_specs=[
            pl.BlockSpec((BK, 1), lambda i, k: (k, 0)),
            pl.BlockSpec((BK, 1), lambda i, k: (k, 0)),
            pl.BlockSpec((BN2, BK), lambda i, k: (i, k)),
            pl.BlockSpec((BN2, BK), lambda i, k: (i, k)),
            pl.BlockSpec((N, 1), lambda i, k: (0, 0)),
        ],
        out_specs=pl.BlockSpec((1, 2), lambda i, k: (0, 0)),
        out_shape=jax.ShapeDtypeStruct((1, 2), jnp.float32),
        scratch_shapes=[
            pltpu.SMEM((1, 2), jnp.float32),
            pltpu.VMEM((BN2, 1), jnp.float32),
        ],
    )(x, y, Q, AT, c2)


def _tc1_kernel(x_ref, y_ref, Q_ref, AT_ref, c_ref, out_ref, acc_ref):
    i = pl.program_id(0)

    @pl.when(i == 0)
    def _init():
        c_full = c_ref[...]
        acc_ref[0, 0] = 0.0
        acc_ref[0, 1] = jnp.sum(c_full * c_full)

    c_blk = c_ref[pl.ds(i * BN2, BN2), :]
    r = (
        jnp.dot(Q_ref[...], x_ref[...], preferred_element_type=jnp.float32)
        + jnp.dot(AT_ref[...], y_ref[...], preferred_element_type=jnp.float32)
        + c_blk
    )
    acc_ref[0, 0] += jnp.sum(r * r)

    @pl.when(i == pl.num_programs(0) - 1)
    def _fin():
        out_ref[...] = jnp.concatenate(
            [
                jnp.full((1, 1), acc_ref[0, 0], dtype=jnp.float32),
                jnp.full((1, 1), acc_ref[0, 1], dtype=jnp.float32),
            ],
            axis=1,
        )


def _tc1_full(Q, AT, c2, x, y):
    return pl.pallas_call(
        _tc1_kernel,
        grid=(N // BN2,),
        in_specs=[
            pl.BlockSpec((K, 1), lambda i: (0, 0)),
            pl.BlockSpec((K, 1), lambda i: (0, 0)),
            pl.BlockSpec((BN2, K), lambda i: (i, 0)),
            pl.BlockSpec((BN2, K), lambda i: (i, 0)),
            pl.BlockSpec((N, 1), lambda i: (0, 0)),
        ],
        out_specs=pl.BlockSpec((1, 2), lambda i: (0, 0)),
        out_shape=jax.ShapeDtypeStruct((1, 2), jnp.float32),
        scratch_shapes=[pltpu.SMEM((1, 2), jnp.float32)],
    )(x, y, Q, AT, c2)


def kernel(Q, AT, b, c, x, y):
    del b
    c2 = c.reshape(N, 1)
    tc_out = _tc1_full(Q, AT, c2, x, y)
    return jnp.sqrt(tc_out[0, 0]) / (0.0001 + jnp.sqrt(tc_out[0, 1]))


# speedup vs baseline: 1.4993x; 1.0992x over previous
import functools

import jax
import jax.numpy as jnp
from jax import lax
from jax.experimental import pallas as pl
from jax.experimental.pallas import tpu as pltpu
from jax.experimental.pallas import tpu_sc as plsc

N = 4096
M = 4096
K = 4096

BN = 256
SC_ROWS = 1024
SC_BLOCKS = SC_ROWS // BN

NC = 2
NS = 16
NW = NC * NS
L = 16
RPW = SC_ROWS // NW
G = 4
NG = RPW // G
KC = K // L


def _tc_kernel(x_ref, y_ref, Q_ref, AT_ref, c_ref, out_ref, acc_ref):
    i = pl.program_id(0)

    @pl.when(i == 0)
    def _init():
        c_full = c_ref[...]
        acc_ref[0, 0] = 0.0
        acc_ref[0, 1] = jnp.sum(c_full * c_full)

    c_blk = c_ref[pl.ds((SC_BLOCKS + i) * BN, BN), :]
    r = (
        jnp.dot(Q_ref[...], x_ref[...], preferred_element_type=jnp.float32)
        + jnp.dot(AT_ref[...], y_ref[...], preferred_element_type=jnp.float32)
        + c_blk
    )
    acc_ref[0, 0] += jnp.sum(r * r)

    @pl.when(i == pl.num_programs(0) - 1)
    def _fin():
        out_ref[...] = jnp.concatenate(
            [
                jnp.full((1, 1), acc_ref[0, 0], dtype=jnp.float32),
                jnp.full((1, 1), acc_ref[0, 1], dtype=jnp.float32),
            ],
            axis=1,
        )


def _tc_partial(Q, AT, c2, x, y):
    n_tc = N - SC_ROWS
    return pl.pallas_call(
        _tc_kernel,
        grid=(n_tc // BN,),
        in_specs=[
            pl.BlockSpec((K, 1), lambda i: (0, 0)),
            pl.BlockSpec((K, 1), lambda i: (0, 0)),
            pl.BlockSpec((BN, K), lambda i: (SC_BLOCKS + i, 0)),
            pl.BlockSpec((BN, K), lambda i: (SC_BLOCKS + i, 0)),
            pl.BlockSpec((N, 1), lambda i: (0, 0)),
        ],
        out_specs=pl.BlockSpec((1, 2), lambda i: (0, 0)),
        out_shape=jax.ShapeDtypeStruct((1, 2), jnp.float32),
        scratch_shapes=[pltpu.SMEM((1, 2), jnp.float32)],
    )(x, y, Q, AT, c2)


def _hsum(v, tmp_ref):
    idx = lax.iota(jnp.int32, L)
    for s in (8, 4, 2, 1):
        tmp_ref[...] = v
        perm = (idx + s) & (L - 1)
        v = v + plsc.load_gather(tmp_ref, [perm])
    return v[0]


def _sc_body(Q_hbm, AT_hbm, c_hbm, x_hbm, y_hbm, out_hbm,
             x_v, y_v, c_v, q_v, a_v, o_v, tmp_v,
             sx, sy, sc_, sq0, sq1, sa0, sa1):
    wid = lax.axis_index("s") * NC + lax.axis_index("c")
    base = wid * RPW
    cp_x = pltpu.async_copy(x_hbm, x_v, sx)
    cp_y = pltpu.async_copy(y_hbm, y_v, sy)
    cp_c = pltpu.async_copy(c_hbm.at[pl.ds(base, RPW)], c_v, sc_)

    q_sems = (sq0, sq1)
    a_sems = (sa0, sa1)

    def start(g):
        buf = g % 2
        row0 = base + g * G
        hq = pltpu.async_copy(Q_hbm.at[pl.ds(row0, G)], q_v.at[buf], q_sems[buf])
        ha = pltpu.async_copy(AT_hbm.at[pl.ds(row0, G)], a_v.at[buf], a_sems[buf])
        return hq, ha

    handles = [None] * NG
    handles[0] = start(0)

    cp_x.wait()
    cp_y.wait()
    cp_c.wait()
    c_lo = c_v[pl.ds(0, L)]
    c_hi = c_v[pl.ds(L, L)]

    acc = jnp.float32(0.0)
    for g in range(NG):
        buf = g % 2
        if g + 1 < NG:
            handles[g + 1] = start(g + 1)
        hq, ha = handles[g]
        hq.wait()
        ha.wait()
        qb = q_v.at[buf]
        ab = a_v.at[buf]

        def body(i, accs):
            o = i * (2 * L)
            new = list(accs)
            for u in range(2):
                oo = o + u * L
                xk = x_v[pl.ds(oo, L)]
                yk = y_v[pl.ds(oo, L)]
                for gg in range(G):
                    new[gg] = new[gg] + qb[gg, pl.ds(oo, L)] * xk
                    new[G + gg] = new[G + gg] + ab[gg, pl.ds(oo, L)] * yk
            return tuple(new)

        zeros = tuple(jnp.zeros((L,), jnp.float32) for _ in range(2 * G))
        accs = lax.fori_loop(0, KC // 2, body, zeros)
        for gg in range(G):
            idx = g * G + gg
            c_val = c_lo[idx] if idx < L else c_hi[idx - L]
            v = _hsum(accs[gg] + accs[G + gg], tmp_v) + c_val
            acc = acc + v * v

    o_v[...] = jnp.full((L,), acc * 0.0625, dtype=jnp.float32)
    pltpu.sync_copy(o_v, out_hbm.at[wid])


def _sc_partial(Q, AT, c1, xf, yf):
    mesh = plsc.VectorSubcoreMesh(core_axis_name="c", subcore_axis_name="s")
    run = pl.kernel(
        _sc_body,
        out_type=jax.ShapeDtypeStruct((NW, L), jnp.float32),
        mesh=mesh,
        scratch_types=[
            pltpu.VMEM((K,), jnp.float32),
            pltpu.VMEM((K,), jnp.float32),
            pltpu.VMEM((RPW,), jnp.float32),
            pltpu.VMEM((2, G, K), jnp.float32),
            pltpu.VMEM((2, G, K), jnp.float32),
            pltpu.VMEM((L,), jnp.float32),
            pltpu.VMEM((L,), jnp.float32),
            pltpu.SemaphoreType.DMA,
            pltpu.SemaphoreType.DMA,
            pltpu.SemaphoreType.DMA,
            pltpu.SemaphoreType.DMA,
            pltpu.SemaphoreType.DMA,
            pltpu.SemaphoreType.DMA,
            pltpu.SemaphoreType.DMA,
        ],
        compiler_params=pltpu.CompilerParams(needs_layout_passes=False),
    )
    return run(Q, AT, c1, xf, yf)


BK = 2048
BN2 = 256


def _tc2_kernel(x_ref, y_ref, Q_ref, AT_ref, c_ref, out_ref, acc_ref, vec_ref):
    i = pl.program_id(0)
    k = pl.program_id(1)

    @pl.when(jnp.logical_and(i == 0, k == 0))
    def _init():
        c_full = c_ref[...]
        acc_ref[0, 0] = 0.0
        acc_ref[0, 1] = jnp.sum(c_full * c_full)

    part = (
        jnp.dot(Q_ref[...], x_ref[...], preferred_element_type=jnp.float32)
        + jnp.dot(AT_ref[...], y_ref[...], preferred_element_type=jnp.float32)
    )

    @pl.when(k == 0)
    def _first():
        vec_ref[...] = part

    @pl.when(k > 0)
    def _rest():
        vec_ref[...] += part

    @pl.when(k == pl.num_programs(1) - 1)
    def _row_done():
        r = vec_ref[...] + c_ref[pl.ds(i * BN2, BN2), :]
        acc_ref[0, 0] += jnp.sum(r * r)

    @pl.when(
        jnp.logical_and(i == pl.num_programs(0) - 1, k == pl.num_programs(1) - 1)
    )
    def _fin():
        out_ref[...] = jnp.concatenate(
            [
                jnp.full((1, 1), acc_ref[0, 0], dtype=jnp.float32),
                jnp.full((1, 1), acc_ref[0, 1], dtype=jnp.float32),
            ],
            axis=1,
        )


def _tc2_full(Q, AT, c2, x, y):
    return pl.pallas_call(
        _tc2_kernel,
        grid=(N // BN2, K // BK),
        in_specs=[
            pl.BlockSpec((BK, 1), lambda i, k: (k, 0)),
            pl.BlockSpec((BK, 1), lambda i, k: (k, 0)),
            pl.BlockSpec((BN2, BK), lambda i, k: (i, k)),
            pl.BlockSpec((BN2, BK), lambda i, k: (i, k)),
            pl.BlockSpec((N, 1), lambda i, k: (0, 0)),
        ],
        out_specs=pl.BlockSpec((1, 2), lambda i, k: (0, 0)),
        out_shape=jax.ShapeDtypeStruct((1, 2), jnp.float32),
        scratch_shapes=[
            pltpu.SMEM((1, 2), jnp.float32),
            pltpu.VMEM((BN2, 1), jnp.float32),
        ],
    )(x, y, Q, AT, c2)


def _tc1_kernel(x_ref, y_ref, Q_ref, AT_ref, c_ref, out_ref, acc_ref):
    i = pl.program_id(0)

    @pl.when(i == 0)
    def _init():
        c_full = c_ref[...]
        acc_ref[0, 0] = 0.0
        acc_ref[0, 1] = jnp.sum(c_full * c_full)

    r = (
        jnp.dot(Q_ref[...], x_ref[...], preferred_element_type=jnp.float32)
        + jnp.dot(AT_ref[...], y_ref[...], preferred_element_type=jnp.float32)
    )
    cr = c_ref[:, pl.ds(i * BN2, BN2)]
    cross = jnp.dot(cr, r, preferred_element_type=jnp.float32)
    acc_ref[0, 0] += jnp.sum(r * r) + 2.0 * cross[0, 0]

    @pl.when(i == pl.num_programs(0) - 1)
    def _fin():
        csq = acc_ref[0, 1]
        top = jnp.sqrt(acc_ref[0, 0] + csq)
        bot = 0.0001 + jnp.sqrt(csq)
        out_ref[...] = jnp.full((1, 1), top / bot, dtype=jnp.float32)


def _tc1_full(Q, AT, c_row, x, y):
    return pl.pallas_call(
        _tc1_kernel,
        grid=(N // BN2,),
        in_specs=[
            pl.BlockSpec((K, 1), lambda i: (0, 0)),
            pl.BlockSpec((K, 1), lambda i: (0, 0)),
            pl.BlockSpec((BN2, K), lambda i: (i, 0)),
            pl.BlockSpec((BN2, K), lambda i: (i, 0)),
            pl.BlockSpec((1, N), lambda i: (0, 0)),
        ],
        out_specs=pl.BlockSpec((1, 1), lambda i: (0, 0)),
        out_shape=jax.ShapeDtypeStruct((1, 1), jnp.float32),
        scratch_shapes=[pltpu.SMEM((1, 2), jnp.float32)],
    )(x, y, Q, AT, c_row)


def kernel(Q, AT, b, c, x, y):
    del b
    out = _tc1_full(Q, AT, c.reshape(1, N), x, y)
    return out[0, 0]


# speedup vs baseline: 1.6641x; 1.1099x over previous
import functools

import jax
import jax.numpy as jnp
from jax import lax
from jax.experimental import pallas as pl
from jax.experimental.pallas import tpu as pltpu
from jax.experimental.pallas import tpu_sc as plsc

N = 4096
M = 4096
K = 4096

BN = 256
SC_ROWS = 1024
SC_BLOCKS = SC_ROWS // BN

NC = 2
NS = 16
NW = NC * NS
L = 16
RPW = SC_ROWS // NW
G = 4
NG = RPW // G
KC = K // L


def _tc_kernel(x_ref, y_ref, Q_ref, AT_ref, c_ref, out_ref, acc_ref):
    i = pl.program_id(0)

    @pl.when(i == 0)
    def _init():
        c_full = c_ref[...]
        acc_ref[0, 0] = 0.0
        acc_ref[0, 1] = jnp.sum(c_full * c_full)

    c_blk = c_ref[pl.ds((SC_BLOCKS + i) * BN, BN), :]
    r = (
        jnp.dot(Q_ref[...], x_ref[...], preferred_element_type=jnp.float32)
        + jnp.dot(AT_ref[...], y_ref[...], preferred_element_type=jnp.float32)
        + c_blk
    )
    acc_ref[0, 0] += jnp.sum(r * r)

    @pl.when(i == pl.num_programs(0) - 1)
    def _fin():
        out_ref[...] = jnp.concatenate(
            [
                jnp.full((1, 1), acc_ref[0, 0], dtype=jnp.float32),
                jnp.full((1, 1), acc_ref[0, 1], dtype=jnp.float32),
            ],
            axis=1,
        )


def _tc_partial(Q, AT, c2, x, y):
    n_tc = N - SC_ROWS
    return pl.pallas_call(
        _tc_kernel,
        grid=(n_tc // BN,),
        in_specs=[
            pl.BlockSpec((K, 1), lambda i: (0, 0)),
            pl.BlockSpec((K, 1), lambda i: (0, 0)),
            pl.BlockSpec((BN, K), lambda i: (SC_BLOCKS + i, 0)),
            pl.BlockSpec((BN, K), lambda i: (SC_BLOCKS + i, 0)),
            pl.BlockSpec((N, 1), lambda i: (0, 0)),
        ],
        out_specs=pl.BlockSpec((1, 2), lambda i: (0, 0)),
        out_shape=jax.ShapeDtypeStruct((1, 2), jnp.float32),
        scratch_shapes=[pltpu.SMEM((1, 2), jnp.float32)],
    )(x, y, Q, AT, c2)


def _hsum(v, tmp_ref):
    idx = lax.iota(jnp.int32, L)
    for s in (8, 4, 2, 1):
        tmp_ref[...] = v
        perm = (idx + s) & (L - 1)
        v = v + plsc.load_gather(tmp_ref, [perm])
    return v[0]


def _sc_body(Q_hbm, AT_hbm, c_hbm, x_hbm, y_hbm, out_hbm,
             x_v, y_v, c_v, q_v, a_v, o_v, tmp_v,
             sx, sy, sc_, sq0, sq1, sa0, sa1):
    wid = lax.axis_index("s") * NC + lax.axis_index("c")
    base = wid * RPW
    cp_x = pltpu.async_copy(x_hbm, x_v, sx)
    cp_y = pltpu.async_copy(y_hbm, y_v, sy)
    cp_c = pltpu.async_copy(c_hbm.at[pl.ds(base, RPW)], c_v, sc_)

    q_sems = (sq0, sq1)
    a_sems = (sa0, sa1)

    def start(g):
        buf = g % 2
        row0 = base + g * G
        hq = pltpu.async_copy(Q_hbm.at[pl.ds(row0, G)], q_v.at[buf], q_sems[buf])
        ha = pltpu.async_copy(AT_hbm.at[pl.ds(row0, G)], a_v.at[buf], a_sems[buf])
        return hq, ha

    handles = [None] * NG
    handles[0] = start(0)

    cp_x.wait()
    cp_y.wait()
    cp_c.wait()
    c_lo = c_v[pl.ds(0, L)]
    c_hi = c_v[pl.ds(L, L)]

    acc = jnp.float32(0.0)
    for g in range(NG):
        buf = g % 2
        if g + 1 < NG:
            handles[g + 1] = start(g + 1)
        hq, ha = handles[g]
        hq.wait()
        ha.wait()
        qb = q_v.at[buf]
        ab = a_v.at[buf]

        def body(i, accs):
            o = i * (2 * L)
            new = list(accs)
            for u in range(2):
                oo = o + u * L
                xk = x_v[pl.ds(oo, L)]
                yk = y_v[pl.ds(oo, L)]
                for gg in range(G):
                    new[gg] = new[gg] + qb[gg, pl.ds(oo, L)] * xk
                    new[G + gg] = new[G + gg] + ab[gg, pl.ds(oo, L)] * yk
            return tuple(new)

        zeros = tuple(jnp.zeros((L,), jnp.float32) for _ in range(2 * G))
        accs = lax.fori_loop(0, KC // 2, body, zeros)
        for gg in range(G):
            idx = g * G + gg
            c_val = c_lo[idx] if idx < L else c_hi[idx - L]
            v = _hsum(accs[gg] + accs[G + gg], tmp_v) + c_val
            acc = acc + v * v

    o_v[...] = jnp.full((L,), acc * 0.0625, dtype=jnp.float32)
    pltpu.sync_copy(o_v, out_hbm.at[wid])


def _sc_partial(Q, AT, c1, xf, yf):
    mesh = plsc.VectorSubcoreMesh(core_axis_name="c", subcore_axis_name="s")
    run = pl.kernel(
        _sc_body,
        out_type=jax.ShapeDtypeStruct((NW, L), jnp.float32),
        mesh=mesh,
        scratch_types=[
            pltpu.VMEM((K,), jnp.float32),
            pltpu.VMEM((K,), jnp.float32),
            pltpu.VMEM((RPW,), jnp.float32),
            pltpu.VMEM((2, G, K), jnp.float32),
            pltpu.VMEM((2, G, K), jnp.float32),
            pltpu.VMEM((L,), jnp.float32),
            pltpu.VMEM((L,), jnp.float32),
            pltpu.SemaphoreType.DMA,
            pltpu.SemaphoreType.DMA,
            pltpu.SemaphoreType.DMA,
            pltpu.SemaphoreType.DMA,
            pltpu.SemaphoreType.DMA,
            pltpu.SemaphoreType.DMA,
            pltpu.SemaphoreType.DMA,
        ],
        compiler_params=pltpu.CompilerParams(needs_layout_passes=False),
    )
    return run(Q, AT, c1, xf, yf)


BK = 2048
BN2 = 256


def _tc2_kernel(x_ref, y_ref, Q_ref, AT_ref, c_ref, out_ref, acc_ref, vec_ref):
    i = pl.program_id(0)
    k = pl.program_id(1)

    @pl.when(jnp.logical_and(i == 0, k == 0))
    def _init():
        c_full = c_ref[...]
        acc_ref[0, 0] = 0.0
        acc_ref[0, 1] = jnp.sum(c_full * c_full)

    part = (
        jnp.dot(Q_ref[...], x_ref[...], preferred_element_type=jnp.float32)
        + jnp.dot(AT_ref[...], y_ref[...], preferred_element_type=jnp.float32)
    )

    @pl.when(k == 0)
    def _first():
        vec_ref[...] = part

    @pl.when(k > 0)
    def _rest():
        vec_ref[...] += part

    @pl.when(k == pl.num_programs(1) - 1)
    def _row_done():
        r = vec_ref[...] + c_ref[pl.ds(i * BN2, BN2), :]
        acc_ref[0, 0] += jnp.sum(r * r)

    @pl.when(
        jnp.logical_and(i == pl.num_programs(0) - 1, k == pl.num_programs(1) - 1)
    )
    def _fin():
        out_ref[...] = jnp.concatenate(
            [
                jnp.full((1, 1), acc_ref[0, 0], dtype=jnp.float32),
                jnp.full((1, 1), acc_ref[0, 1], dtype=jnp.float32),
            ],
            axis=1,
        )


def _tc2_full(Q, AT, c2, x, y):
    return pl.pallas_call(
        _tc2_kernel,
        grid=(N // BN2, K // BK),
        in_specs=[
            pl.BlockSpec((BK, 1), lambda i, k: (k, 0)),
            pl.BlockSpec((BK, 1), lambda i, k: (k, 0)),
            pl.BlockSpec((BN2, BK), lambda i, k: (i, k)),
            pl.BlockSpec((BN2, BK), lambda i, k: (i, k)),
            pl.BlockSpec((N, 1), lambda i, k: (0, 0)),
        ],
        out_specs=pl.BlockSpec((1, 2), lambda i, k: (0, 0)),
        out_shape=jax.ShapeDtypeStruct((1, 2), jnp.float32),
        scratch_shapes=[
            pltpu.SMEM((1, 2), jnp.float32),
            pltpu.VMEM((BN2, 1), jnp.float32),
        ],
    )(x, y, Q, AT, c2)


def _tc1_kernel(x_ref, y_ref, Q_ref, AT_ref, c_ref, out_ref, acc_ref,
                xcol_ref, ycol_ref):
    i = pl.program_id(0)

    @pl.when(i == 0)
    def _init():
        c_full = c_ref[...]
        acc_ref[0, 0] = 0.0
        acc_ref[0, 1] = jnp.sum(c_full * c_full)
        xcol_ref[...] = jnp.transpose(x_ref[...])
        ycol_ref[...] = jnp.transpose(y_ref[...])

    r = (
        jnp.dot(Q_ref[...], xcol_ref[...], preferred_element_type=jnp.float32)
        + jnp.dot(AT_ref[...], ycol_ref[...], preferred_element_type=jnp.float32)
    )
    cr = c_ref[:, pl.ds(i * BN2, BN2)]
    cross = jnp.dot(cr, r, preferred_element_type=jnp.float32)
    acc_ref[0, 0] += jnp.sum(r * r) + 2.0 * cross[0, 0]

    @pl.when(i == pl.num_programs(0) - 1)
    def _fin():
        csq = acc_ref[0, 1]
        top = jnp.sqrt(acc_ref[0, 0] + csq)
        bot = 0.0001 + jnp.sqrt(csq)
        out_ref[...] = jnp.full((1, 1), top / bot, dtype=jnp.float32)


def _tc1_full(Q, AT, c_row, x, y):
    return pl.pallas_call(
        _tc1_kernel,
        grid=(N // BN2,),
        in_specs=[
            pl.BlockSpec((1, K), lambda i: (0, 0)),
            pl.BlockSpec((1, K), lambda i: (0, 0)),
            pl.BlockSpec((BN2, K), lambda i: (i, 0)),
            pl.BlockSpec((BN2, K), lambda i: (i, 0)),
            pl.BlockSpec((1, N), lambda i: (0, 0)),
        ],
        out_specs=pl.BlockSpec((1, 1), lambda i: (0, 0)),
        out_shape=jax.ShapeDtypeStruct((1, 1), jnp.float32),
        scratch_shapes=[
            pltpu.SMEM((1, 2), jnp.float32),
            pltpu.VMEM((K, 1), jnp.float32),
            pltpu.VMEM((K, 1), jnp.float32),
        ],
    )(x, y, Q, AT, c_row)


def kernel(Q, AT, b, c, x, y):
    del b
    out = _tc1_full(
        Q, AT, c.reshape(1, N), x.reshape(1, K), y.reshape(1, K)
    )
    return out[0, 0]


# speedup vs baseline: 1.8215x; 1.0945x over previous
import functools

import jax
import jax.numpy as jnp
from jax import lax
from jax.experimental import pallas as pl
from jax.experimental.pallas import tpu as pltpu
from jax.experimental.pallas import tpu_sc as plsc

N = 4096
M = 4096
K = 4096

BN = 256
SC_ROWS = 1024
SC_BLOCKS = SC_ROWS // BN

NC = 2
NS = 16
NW = NC * NS
L = 16
RPW = SC_ROWS // NW
G = 4
NG = RPW // G
KC = K // L


def _tc_kernel(x_ref, y_ref, Q_ref, AT_ref, c_ref, out_ref, acc_ref):
    i = pl.program_id(0)

    @pl.when(i == 0)
    def _init():
        c_full = c_ref[...]
        acc_ref[0, 0] = 0.0
        acc_ref[0, 1] = jnp.sum(c_full * c_full)

    c_blk = c_ref[pl.ds((SC_BLOCKS + i) * BN, BN), :]
    r = (
        jnp.dot(Q_ref[...], x_ref[...], preferred_element_type=jnp.float32)
        + jnp.dot(AT_ref[...], y_ref[...], preferred_element_type=jnp.float32)
        + c_blk
    )
    acc_ref[0, 0] += jnp.sum(r * r)

    @pl.when(i == pl.num_programs(0) - 1)
    def _fin():
        out_ref[...] = jnp.concatenate(
            [
                jnp.full((1, 1), acc_ref[0, 0], dtype=jnp.float32),
                jnp.full((1, 1), acc_ref[0, 1], dtype=jnp.float32),
            ],
            axis=1,
        )


def _tc_partial(Q, AT, c2, x, y):
    n_tc = N - SC_ROWS
    return pl.pallas_call(
        _tc_kernel,
        grid=(n_tc // BN,),
        in_specs=[
            pl.BlockSpec((K, 1), lambda i: (0, 0)),
            pl.BlockSpec((K, 1), lambda i: (0, 0)),
            pl.BlockSpec((BN, K), lambda i: (SC_BLOCKS + i, 0)),
            pl.BlockSpec((BN, K), lambda i: (SC_BLOCKS + i, 0)),
            pl.BlockSpec((N, 1), lambda i: (0, 0)),
        ],
        out_specs=pl.BlockSpec((1, 2), lambda i: (0, 0)),
        out_shape=jax.ShapeDtypeStruct((1, 2), jnp.float32),
        scratch_shapes=[pltpu.SMEM((1, 2), jnp.float32)],
    )(x, y, Q, AT, c2)


def _hsum(v, tmp_ref):
    idx = lax.iota(jnp.int32, L)
    for s in (8, 4, 2, 1):
        tmp_ref[...] = v
        perm = (idx + s) & (L - 1)
        v = v + plsc.load_gather(tmp_ref, [perm])
    return v[0]


def _sc_body(Q_hbm, AT_hbm, c_hbm, x_hbm, y_hbm, out_hbm,
             x_v, y_v, c_v, q_v, a_v, o_v, tmp_v,
             sx, sy, sc_, sq0, sq1, sa0, sa1):
    wid = lax.axis_index("s") * NC + lax.axis_index("c")
    base = wid * RPW
    cp_x = pltpu.async_copy(x_hbm, x_v, sx)
    cp_y = pltpu.async_copy(y_hbm, y_v, sy)
    cp_c = pltpu.async_copy(c_hbm.at[pl.ds(base, RPW)], c_v, sc_)

    q_sems = (sq0, sq1)
    a_sems = (sa0, sa1)

    def start(g):
        buf = g % 2
        row0 = base + g * G
        hq = pltpu.async_copy(Q_hbm.at[pl.ds(row0, G)], q_v.at[buf], q_sems[buf])
        ha = pltpu.async_copy(AT_hbm.at[pl.ds(row0, G)], a_v.at[buf], a_sems[buf])
        return hq, ha

    handles = [None] * NG
    handles[0] = start(0)

    cp_x.wait()
    cp_y.wait()
    cp_c.wait()
    c_lo = c_v[pl.ds(0, L)]
    c_hi = c_v[pl.ds(L, L)]

    acc = jnp.float32(0.0)
    for g in range(NG):
        buf = g % 2
        if g + 1 < NG:
            handles[g + 1] = start(g + 1)
        hq, ha = handles[g]
        hq.wait()
        ha.wait()
        qb = q_v.at[buf]
        ab = a_v.at[buf]

        def body(i, accs):
            o = i * (2 * L)
            new = list(accs)
            for u in range(2):
                oo = o + u * L
                xk = x_v[pl.ds(oo, L)]
                yk = y_v[pl.ds(oo, L)]
                for gg in range(G):
                    new[gg] = new[gg] + qb[gg, pl.ds(oo, L)] * xk
                    new[G + gg] = new[G + gg] + ab[gg, pl.ds(oo, L)] * yk
            return tuple(new)

        zeros = tuple(jnp.zeros((L,), jnp.float32) for _ in range(2 * G))
        accs = lax.fori_loop(0, KC // 2, body, zeros)
        for gg in range(G):
            idx = g * G + gg
            c_val = c_lo[idx] if idx < L else c_hi[idx - L]
            v = _hsum(accs[gg] + accs[G + gg], tmp_v) + c_val
            acc = acc + v * v

    o_v[...] = jnp.full((L,), acc * 0.0625, dtype=jnp.float32)
    pltpu.sync_copy(o_v, out_hbm.at[wid])


def _sc_partial(Q, AT, c1, xf, yf):
    mesh = plsc.VectorSubcoreMesh(core_axis_name="c", subcore_axis_name="s")
    run = pl.kernel(
        _sc_body,
        out_type=jax.ShapeDtypeStruct((NW, L), jnp.float32),
        mesh=mesh,
        scratch_types=[
            pltpu.VMEM((K,), jnp.float32),
            pltpu.VMEM((K,), jnp.float32),
            pltpu.VMEM((RPW,), jnp.float32),
            pltpu.VMEM((2, G, K), jnp.float32),
            pltpu.VMEM((2, G, K), jnp.float32),
            pltpu.VMEM((L,), jnp.float32),
            pltpu.VMEM((L,), jnp.float32),
            pltpu.SemaphoreType.DMA,
            pltpu.SemaphoreType.DMA,
            pltpu.SemaphoreType.DMA,
            pltpu.SemaphoreType.DMA,
            pltpu.SemaphoreType.DMA,
            pltpu.SemaphoreType.DMA,
            pltpu.SemaphoreType.DMA,
        ],
        compiler_params=pltpu.CompilerParams(needs_layout_passes=False),
    )
    return run(Q, AT, c1, xf, yf)


BK = 2048
BN2 = 256


def _tc2_kernel(x_ref, y_ref, Q_ref, AT_ref, c_ref, out_ref, acc_ref, vec_ref):
    i = pl.program_id(0)
    k = pl.program_id(1)

    @pl.when(jnp.logical_and(i == 0, k == 0))
    def _init():
        c_full = c_ref[...]
        acc_ref[0, 0] = 0.0
        acc_ref[0, 1] = jnp.sum(c_full * c_full)

    part = (
        jnp.dot(Q_ref[...], x_ref[...], preferred_element_type=jnp.float32)
        + jnp.dot(AT_ref[...], y_ref[...], preferred_element_type=jnp.float32)
    )

    @pl.when(k == 0)
    def _first():
        vec_ref[...] = part

    @pl.when(k > 0)
    def _rest():
        vec_ref[...] += part

    @pl.when(k == pl.num_programs(1) - 1)
    def _row_done():
        r = vec_ref[...] + c_ref[pl.ds(i * BN2, BN2), :]
        acc_ref[0, 0] += jnp.sum(r * r)

    @pl.when(
        jnp.logical_and(i == pl.num_programs(0) - 1, k == pl.num_programs(1) - 1)
    )
    def _fin():
        out_ref[...] = jnp.concatenate(
            [
                jnp.full((1, 1), acc_ref[0, 0], dtype=jnp.float32),
                jnp.full((1, 1), acc_ref[0, 1], dtype=jnp.float32),
            ],
            axis=1,
        )


def _tc2_full(Q, AT, c2, x, y):
    return pl.pallas_call(
        _tc2_kernel,
        grid=(N // BN2, K // BK),
        in_specs=[
            pl.BlockSpec((BK, 1), lambda i, k: (k, 0)),
            pl.BlockSpec((BK, 1), lambda i, k: (k, 0)),
            pl.BlockSpec((BN2, BK), lambda i, k: (i, k)),
            pl.BlockSpec((BN2, BK), lambda i, k: (i, k)),
            pl.BlockSpec((N, 1), lambda i, k: (0, 0)),
        ],
        out_specs=pl.BlockSpec((1, 2), lambda i, k: (0, 0)),
        out_shape=jax.ShapeDtypeStruct((1, 2), jnp.float32),
        scratch_shapes=[
            pltpu.SMEM((1, 2), jnp.float32),
            pltpu.VMEM((BN2, 1), jnp.float32),
        ],
    )(x, y, Q, AT, c2)


def _tc1_kernel(x_ref, y_ref, Q_ref, AT_ref, c_ref, out_ref, acc_ref):
    i = pl.program_id(0)

    @pl.when(i == 0)
    def _init():
        c_full = c_ref[...]
        acc_ref[0, 0] = 0.0
        acc_ref[0, 1] = jnp.sum(c_full * c_full)

    dn = (((1,), (1,)), ((), ()))
    r = (
        lax.dot_general(x_ref[...], Q_ref[...], dn,
                        preferred_element_type=jnp.float32)
        + lax.dot_general(y_ref[...], AT_ref[...], dn,
                          preferred_element_type=jnp.float32)
        + c_ref[:, pl.ds(i * BN2, BN2)]
    )
    acc_ref[0, 0] += jnp.sum(r * r)

    @pl.when(i == pl.num_programs(0) - 1)
    def _fin():
        top = jnp.sqrt(acc_ref[0, 0])
        bot = 0.0001 + jnp.sqrt(acc_ref[0, 1])
        out_ref[...] = jnp.full((1, 1), top / bot, dtype=jnp.float32)


def _tc1_full(Q, AT, c_row, x, y):
    return pl.pallas_call(
        _tc1_kernel,
        grid=(N // BN2,),
        in_specs=[
            pl.BlockSpec((1, K), lambda i: (0, 0)),
            pl.BlockSpec((1, K), lambda i: (0, 0)),
            pl.BlockSpec((BN2, K), lambda i: (i, 0)),
            pl.BlockSpec((BN2, K), lambda i: (i, 0)),
            pl.BlockSpec((1, N), lambda i: (0, 0)),
        ],
        out_specs=pl.BlockSpec((1, 1), lambda i: (0, 0)),
        out_shape=jax.ShapeDtypeStruct((1, 1), jnp.float32),
        scratch_shapes=[pltpu.SMEM((1, 2), jnp.float32)],
    )(x, y, Q, AT, c_row)


def kernel(Q, AT, b, c, x, y):
    del b
    out = _tc1_full(
        Q, AT, c.reshape(1, N), x.reshape(1, K), y.reshape(1, K)
    )
    return out[0, 0]
